# baseline (speedup 1.0000x reference)
import functools

import jax
import jax.numpy as jnp
from jax import lax
from jax.experimental import pallas as pl
from jax.experimental.pallas import tpu as pltpu
from jax.experimental.pallas import tpu_sc as plsc

N = 16384
EPS = 1e-12
L = 16
NUM_CORES = 2
NUM_SUBCORES = 16
NUM_WORKERS = NUM_CORES * NUM_SUBCORES
BLOCK = 8192
UNROLL = 8


def _sc_partial_spmv(preds, vals, rows, cols, aux_vals, aux_rows, aux_cols,
                     e_per_w, nfull, part):
    mesh = plsc.VectorSubcoreMesh(core_axis_name="c", subcore_axis_name="s")

    @functools.partial(
        pl.kernel,
        out_type=jax.ShapeDtypeStruct((NUM_WORKERS, N), jnp.float32),
        mesh=mesh,
        compiler_params=pltpu.CompilerParams(needs_layout_passes=False),
        scratch_types=[
            pltpu.VMEM((N,), jnp.float32),
            pltpu.VMEM((N,), jnp.float32),
            pltpu.VMEM((BLOCK,), jnp.float32),
            pltpu.VMEM((BLOCK,), jnp.int32),
            pltpu.VMEM((BLOCK,), jnp.int32),
            pltpu.VMEM((BLOCK,), jnp.float32),
            pltpu.VMEM((BLOCK,), jnp.int32),
            pltpu.VMEM((BLOCK,), jnp.int32),
            pltpu.VMEM((BLOCK,), jnp.float32),
            pltpu.VMEM((BLOCK,), jnp.int32),
            pltpu.VMEM((BLOCK,), jnp.int32),
            pltpu.VMEM((part, ), jnp.float32),
            pltpu.VMEM((part, ), jnp.int32),
            pltpu.VMEM((part, ), jnp.int32),
            pltpu.SemaphoreType.DMA,
            pltpu.SemaphoreType.DMA,
            pltpu.SemaphoreType.DMA,
            pltpu.SemaphoreType.DMA,
            pltpu.SemaphoreType.DMA,
        ],
    )
    def k(preds_hbm, vals_hbm, rows_hbm, cols_hbm,
          aux_vals_hbm, aux_rows_hbm, aux_cols_hbm, out_hbm,
          preds_v, ax_v, vals0, rows0, cols0, vals1, rows1, cols1,
          vals2, rows2, cols2, valsp, rowsp, colsp,
          sem0, sem1, sem2, semp, psem):
        wid = lax.axis_index("s") * NUM_CORES + lax.axis_index("c")
        bufs = ((vals0, rows0, cols0, sem0), (vals1, rows1, cols1, sem1),
                (vals2, rows2, cols2, sem2))
        pbuf = (valsp, rowsp, colsp, semp)

        def start_chunk(base, buf, n):
            vb, rb, cb, sem = buf
            pltpu.async_copy(vals_hbm.at[pl.ds(base, n)], vb, sem)
            pltpu.async_copy(rows_hbm.at[pl.ds(base, n)], rb, sem)
            pltpu.async_copy(cols_hbm.at[pl.ds(base, n)], cb, sem)

        def drain_chunk(buf, n):
            vb, rb, cb, sem = buf
            pltpu.make_async_copy(vals_hbm.at[pl.ds(0, n)], vb, sem).wait()
            pltpu.make_async_copy(rows_hbm.at[pl.ds(0, n)], rb, sem).wait()
            pltpu.make_async_copy(cols_hbm.at[pl.ds(0, n)], cb, sem).wait()

        lane = lax.iota(jnp.int32, L)
        shift_idx = jnp.minimum(lane + 1, L - 1)
        is_last = lane == (L - 1)
        not_last = lane < (L - 1)
        gdn = lax.GatherDimensionNumbers(
            offset_dims=(), collapsed_slice_dims=(0,), start_index_map=(0,))

        def process(buf, n):
            vb, rb, cb, _ = buf

            @plsc.parallel_loop(0, n // L, 1, unroll=UNROLL)
            def _(j):
                off = j * L
                v = vb[pl.ds(off, L)]
                r = rb[pl.ds(off, L)]
                c = cb[pl.ds(off, L)]
                p = plsc.load_gather(preds_v, [c])
                cs = plsc.cumsum(v * p)
                r_next = lax.gather(
                    r, shift_idx[:, None], gdn, slice_sizes=(1,),
                    mode=lax.GatherScatterMode.PROMISE_IN_BOUNDS)
                end = (r != r_next) | is_last
                plsc.addupdate_scatter(ax_v, [r], cs, mask=end)
                plsc.addupdate_scatter(ax_v, [r_next], -cs,
                                       mask=end & not_last)

        e0 = wid * e_per_w
        for b in range(min(3, nfull)):
            start_chunk(e0 + b * BLOCK, bufs[b], BLOCK)

        @pl.when(wid < NUM_WORKERS - 1)
        def _():
            vb, rb, cb, sem = pbuf
            base = e0 + nfull * BLOCK
            pltpu.async_copy(vals_hbm.at[pl.ds(base, part)], vb, sem)
            pltpu.async_copy(rows_hbm.at[pl.ds(base, part)], rb, sem)
            pltpu.async_copy(cols_hbm.at[pl.ds(base, part)], cb, sem)

        @pl.when(wid == NUM_WORKERS - 1)
        def _():
            vb, rb, cb, sem = pbuf
            pltpu.async_copy(aux_vals_hbm, vb, sem)
            pltpu.async_copy(aux_rows_hbm, rb, sem)
            pltpu.async_copy(aux_cols_hbm, cb, sem)

        pcopy = pltpu.async_copy(preds_hbm, preds_v, psem)

        @plsc.parallel_loop(0, N // L, 1, unroll=UNROLL)
        def _(i):
            ax_v[pl.ds(i * L, L)] = jnp.zeros((L,), jnp.float32)

        pcopy.wait()

        for b in range(nfull):
            drain_chunk(bufs[b % 3], BLOCK)
            process(bufs[b % 3], BLOCK)
            if b + 3 < nfull:
                start_chunk(e0 + (b + 3) * BLOCK, bufs[b % 3], BLOCK)

        drain_chunk(pbuf, part)
        process(pbuf, part)

        pltpu.sync_copy(ax_v, out_hbm.at[wid])

    return k(preds, vals, rows, cols, aux_vals, aux_rows, aux_cols)


def _finish(partials, target):
    def body(p_ref, t_ref, o_ref):
        ax = jnp.sum(p_ref[...], axis=0)
        t = t_ref[...]
        res = t - ax
        ss_res = jnp.sum(res * res)
        ss_t = jnp.sum(t * t)
        val = jnp.sqrt(ss_res) / (jnp.sqrt(ss_t) + EPS)
        o_ref[...] = jnp.full((1, 1), val, jnp.float32)

    return pl.pallas_call(
        body,
        out_shape=jax.ShapeDtypeStruct((1, 1), jnp.float32),
    )(partials, target)


def kernel(preds, target_b, matrix_vals, matrix_rows, matrix_cols, batch_map):
    nnz = matrix_vals.shape[0]
    e_per_w = -(-nnz // (NUM_WORKERS * L)) * L
    nfull = e_per_w // BLOCK
    part = e_per_w - nfull * BLOCK
    while part < NUM_WORKERS * L and nfull > 0:
        nfull -= 1
        part += BLOCK
    s_last = NUM_WORKERS * e_per_w - part
    real = nnz - s_last
    aux_vals = jnp.zeros((part,), jnp.float32)
    aux_rows = jnp.full((part,), N - 1, jnp.int32)
    aux_cols = jnp.zeros((part,), jnp.int32)
    aux_vals = aux_vals.at[:real].set(matrix_vals[s_last:])
    aux_rows = aux_rows.at[:real].set(matrix_rows[s_last:])
    aux_cols = aux_cols.at[:real].set(matrix_cols[s_last:])
    partials = _sc_partial_spmv(preds, matrix_vals, matrix_rows, matrix_cols,
                                aux_vals, aux_rows, aux_cols,
                                e_per_w, nfull, part)
    out = _finish(partials.reshape(NUM_WORKERS, 128, 128),
                  target_b.reshape(128, 128))
    return out[0, 0]


# speedup vs baseline: 1.0169x; 1.0169x over previous
import functools

import jax
import jax.numpy as jnp
from jax import lax
from jax.experimental import pallas as pl
from jax.experimental.pallas import tpu as pltpu
from jax.experimental.pallas import tpu_sc as plsc

N = 16384
ROW_BITS = 14
ROW_MASK = (1 << ROW_BITS) - 1
EPS = 1e-12
L = 16
NUM_CORES = 2
NUM_SUBCORES = 16
NUM_WORKERS = NUM_CORES * NUM_SUBCORES
BLOCK = 8192
VPB = BLOCK // L
UNROLL = 8


def _sc_partial_spmv(preds, vals, rows, cols, aux_vals, aux_rows, aux_cols,
                     full):
    mesh = plsc.VectorSubcoreMesh(core_axis_name="c", subcore_axis_name="s")
    nblocks = full + 1

    @functools.partial(
        pl.kernel,
        out_type=jax.ShapeDtypeStruct((NUM_WORKERS, N), jnp.float32),
        mesh=mesh,
        compiler_params=pltpu.CompilerParams(needs_layout_passes=False, disable_bounds_checks=True),
        scratch_types=[
            pltpu.VMEM((N,), jnp.float32),
            pltpu.VMEM((N,), jnp.float32),
            pltpu.VMEM((BLOCK,), jnp.float32),
            pltpu.VMEM((BLOCK,), jnp.int32),
            pltpu.VMEM((BLOCK,), jnp.int32),
            pltpu.VMEM((BLOCK,), jnp.float32),
            pltpu.VMEM((BLOCK,), jnp.int32),
            pltpu.VMEM((BLOCK,), jnp.int32),
            pltpu.SemaphoreType.DMA,
            pltpu.SemaphoreType.DMA,
            pltpu.SemaphoreType.DMA,
        ],
    )
    def k(preds_hbm, vals_hbm, rows_hbm, cols_hbm,
          aux_vals_hbm, aux_rows_hbm, aux_cols_hbm, out_hbm,
          preds_v, ax_v, vals0, rows0, cols0, vals1, rows1, cols1,
          sem0, sem1, psem):
        wid = lax.axis_index("s") * NUM_CORES + lax.axis_index("c")
        bufs = ((vals0, rows0, cols0, sem0), (vals1, rows1, cols1, sem1))

        def start_block(bi, buf):
            vb, rb, cb, sem = buf

            @pl.when(bi < full)
            def _():
                base = bi * BLOCK
                pltpu.async_copy(vals_hbm.at[pl.ds(base, BLOCK)], vb, sem)
                pltpu.async_copy(rows_hbm.at[pl.ds(base, BLOCK)], rb, sem)
                pltpu.async_copy(cols_hbm.at[pl.ds(base, BLOCK)], cb, sem)

            @pl.when(bi >= full)
            def _():
                abase = jnp.minimum(bi - full, 1) * BLOCK
                pltpu.async_copy(aux_vals_hbm.at[pl.ds(abase, BLOCK)], vb, sem)
                pltpu.async_copy(aux_rows_hbm.at[pl.ds(abase, BLOCK)], rb, sem)
                pltpu.async_copy(aux_cols_hbm.at[pl.ds(abase, BLOCK)], cb, sem)

        def drain_block(buf):
            vb, rb, cb, sem = buf
            pltpu.make_async_copy(vals_hbm.at[pl.ds(0, BLOCK)], vb, sem).wait()
            pltpu.make_async_copy(rows_hbm.at[pl.ds(0, BLOCK)], rb, sem).wait()
            pltpu.make_async_copy(cols_hbm.at[pl.ds(0, BLOCK)], cb, sem).wait()

        lane = lax.iota(jnp.int32, L)
        shift_idx = jnp.minimum(lane + 1, L - 1)
        is_last = lane == (L - 1)
        not_last = lane < (L - 1)
        gdn = lax.GatherDimensionNumbers(
            offset_dims=(), collapsed_slice_dims=(0,), start_index_map=(0,))

        def process(buf):
            vb, rb, cb, _ = buf

            @plsc.parallel_loop(0, VPB, 1, unroll=UNROLL)
            def _(j):
                off = j * L
                v = vb[pl.ds(off, L)]
                r = rb[pl.ds(off, L)]
                c = cb[pl.ds(off, L)]
                p = plsc.load_gather(preds_v, [c])
                cs = plsc.cumsum(v * p)
                r_next = lax.gather(
                    r, shift_idx[:, None], gdn, slice_sizes=(1,),
                    mode=lax.GatherScatterMode.PROMISE_IN_BOUNDS)
                end = (r != r_next) | is_last
                plsc.addupdate_scatter(ax_v, [r], cs, mask=end)
                plsc.addupdate_scatter(ax_v, [r_next], -cs,
                                       mask=end & not_last)

        bi0 = wid * nblocks // NUM_WORKERS
        nb_w = (wid + 1) * nblocks // NUM_WORKERS - bi0
        start_block(bi0, bufs[0])
        pcopy = pltpu.async_copy(preds_hbm, preds_v, psem)

        @plsc.parallel_loop(0, N // L, 1, unroll=UNROLL)
        def _(i):
            ax_v[pl.ds(i * L, L)] = jnp.zeros((L,), jnp.float32)

        pcopy.wait()

        def body(b, carry):
            nxt = bi0 + b + 1

            @pl.when(b % 2 == 0)
            def _():
                start_block(nxt, bufs[1])
                drain_block(bufs[0])
                process(bufs[0])

            @pl.when(b % 2 == 1)
            def _():
                start_block(nxt, bufs[0])
                drain_block(bufs[1])
                process(bufs[1])

            return carry

        lax.fori_loop(0, nb_w, body, 0)

        @pl.when(nb_w % 2 == 0)
        def _():
            drain_block(bufs[0])

        @pl.when(nb_w % 2 == 1)
        def _():
            drain_block(bufs[1])

        pltpu.sync_copy(ax_v, out_hbm.at[wid])

    return k(preds, vals, rows, cols, aux_vals, aux_rows, aux_cols)


def _finish(partials, target):
    def body(p_ref, t_ref, o_ref):
        ax = jnp.sum(p_ref[...], axis=0)
        t = t_ref[...]
        res = t - ax
        ss_res = jnp.sum(res * res)
        ss_t = jnp.sum(t * t)
        val = jnp.sqrt(ss_res) / (jnp.sqrt(ss_t) + EPS)
        o_ref[...] = jnp.full((1, 1), val, jnp.float32)

    return pl.pallas_call(
        body,
        out_shape=jax.ShapeDtypeStruct((1, 1), jnp.float32),
    )(partials, target)


def kernel(preds, target_b, matrix_vals, matrix_rows, matrix_cols, batch_map):
    nnz = matrix_vals.shape[0]
    full = nnz // BLOCK
    tail = nnz - full * BLOCK
    aux_vals = jnp.zeros((2 * BLOCK,), jnp.float32)
    aux_rows = jnp.full((2 * BLOCK,), N - 1, jnp.int32)
    aux_cols = jnp.zeros((2 * BLOCK,), jnp.int32)
    if tail:
        aux_vals = aux_vals.at[:tail].set(matrix_vals[full * BLOCK:])
        aux_rows = aux_rows.at[:tail].set(matrix_rows[full * BLOCK:])
        aux_cols = aux_cols.at[:tail].set(matrix_cols[full * BLOCK:])
    partials = _sc_partial_spmv(preds, matrix_vals, matrix_rows, matrix_cols,
                                aux_vals, aux_rows, aux_cols, full)
    out = _finish(partials.reshape(NUM_WORKERS, 128, 128),
                  target_b.reshape(128, 128))
    return out[0, 0]


# speedup vs baseline: 1.0645x; 1.0468x over previous
import functools

import jax
import jax.numpy as jnp
from jax import lax
from jax.experimental import pallas as pl
from jax.experimental.pallas import tpu as pltpu
from jax.experimental.pallas import tpu_sc as plsc

N = 16384
ROW_BITS = 14
ROW_MASK = (1 << ROW_BITS) - 1
EPS = 1e-12
L = 16
NUM_CORES = 2
NUM_SUBCORES = 16
NUM_WORKERS = NUM_CORES * NUM_SUBCORES
BLOCK = 8192
VPB = BLOCK // L
UNROLL = 8


def _sc_partial_spmv(preds, vals, rows, cols, aux_vals, aux_rows, aux_cols,
                     full):
    mesh = plsc.VectorSubcoreMesh(core_axis_name="c", subcore_axis_name="s")
    nblocks = full + 1

    @functools.partial(
        pl.kernel,
        out_type=jax.ShapeDtypeStruct((NUM_WORKERS, N), jnp.float32),
        mesh=mesh,
        compiler_params=pltpu.CompilerParams(needs_layout_passes=False, disable_bounds_checks=True),
        scratch_types=[
            pltpu.VMEM((N,), jnp.float32),
            pltpu.VMEM((N,), jnp.float32),
            pltpu.VMEM((BLOCK,), jnp.float32),
            pltpu.VMEM((BLOCK,), jnp.int32),
            pltpu.VMEM((BLOCK,), jnp.int32),
            pltpu.VMEM((BLOCK,), jnp.float32),
            pltpu.VMEM((BLOCK,), jnp.int32),
            pltpu.VMEM((BLOCK,), jnp.int32),
            pltpu.SemaphoreType.DMA,
            pltpu.SemaphoreType.DMA,
            pltpu.SemaphoreType.DMA,
        ],
    )
    def k(preds_hbm, vals_hbm, rows_hbm, cols_hbm,
          aux_vals_hbm, aux_rows_hbm, aux_cols_hbm, out_hbm,
          preds_v, ax_v, vals0, rows0, cols0, vals1, rows1, cols1,
          sem0, sem1, psem):
        wid = lax.axis_index("s") * NUM_CORES + lax.axis_index("c")
        bufs = ((vals0, rows0, cols0, sem0), (vals1, rows1, cols1, sem1))

        def start_block(bi, buf):
            vb, rb, cb, sem = buf

            @pl.when(bi < full)
            def _():
                base = bi * BLOCK
                pltpu.async_copy(vals_hbm.at[pl.ds(base, BLOCK)], vb, sem)
                pltpu.async_copy(rows_hbm.at[pl.ds(base, BLOCK)], rb, sem)
                pltpu.async_copy(cols_hbm.at[pl.ds(base, BLOCK)], cb, sem)

            @pl.when(bi >= full)
            def _():
                abase = jnp.minimum(bi - full, 1) * BLOCK
                pltpu.async_copy(aux_vals_hbm.at[pl.ds(abase, BLOCK)], vb, sem)
                pltpu.async_copy(aux_rows_hbm.at[pl.ds(abase, BLOCK)], rb, sem)
                pltpu.async_copy(aux_cols_hbm.at[pl.ds(abase, BLOCK)], cb, sem)

        def drain_block(buf):
            vb, rb, cb, sem = buf
            pltpu.make_async_copy(vals_hbm.at[pl.ds(0, BLOCK)], vb, sem).wait()
            pltpu.make_async_copy(rows_hbm.at[pl.ds(0, BLOCK)], rb, sem).wait()
            pltpu.make_async_copy(cols_hbm.at[pl.ds(0, BLOCK)], cb, sem).wait()

        lane = lax.iota(jnp.int32, L)
        shift_idx = jnp.minimum(lane + 1, L - 1)
        is_last = lane == (L - 1)
        not_last = lane < (L - 1)
        gdn = lax.GatherDimensionNumbers(
            offset_dims=(), collapsed_slice_dims=(0,), start_index_map=(0,))

        def process(buf):
            vb, rb, cb, _ = buf

            @plsc.parallel_loop(0, VPB, 1, unroll=UNROLL)
            def _(j):
                off = j * L
                v = vb[pl.ds(off, L)]
                r = rb[pl.ds(off, L)]
                c = cb[pl.ds(off, L)]
                p = plsc.load_gather(preds_v, [c])
                cs = plsc.cumsum(v * p)
                r_next = lax.gather(
                    r, shift_idx[:, None], gdn, slice_sizes=(1,),
                    mode=lax.GatherScatterMode.PROMISE_IN_BOUNDS)
                end = (r != r_next) | is_last
                plsc.addupdate_scatter(ax_v, [r], cs, mask=end)
                plsc.addupdate_scatter(ax_v, [r_next], -cs,
                                       mask=end & not_last)

        bi0 = wid * nblocks // NUM_WORKERS
        nb_w = (wid + 1) * nblocks // NUM_WORKERS - bi0
        start_block(bi0, bufs[0])
        pcopy = pltpu.async_copy(preds_hbm, preds_v, psem)

        @plsc.parallel_loop(0, N // L, 1, unroll=UNROLL)
        def _(i):
            ax_v[pl.ds(i * L, L)] = jnp.zeros((L,), jnp.float32)

        pcopy.wait()

        def body(b, carry):
            nxt = bi0 + b + 1

            @pl.when(b % 2 == 0)
            def _():
                start_block(nxt, bufs[1])
                drain_block(bufs[0])
                process(bufs[0])

            @pl.when(b % 2 == 1)
            def _():
                start_block(nxt, bufs[0])
                drain_block(bufs[1])
                process(bufs[1])

            return carry

        lax.fori_loop(0, nb_w, body, 0)

        @pl.when(nb_w % 2 == 0)
        def _():
            drain_block(bufs[0])

        @pl.when(nb_w % 2 == 1)
        def _():
            drain_block(bufs[1])

        pltpu.sync_copy(ax_v, out_hbm.at[wid])

    return k(preds, vals, rows, cols, aux_vals, aux_rows, aux_cols)


def _finish(partials, target):
    def body(p_hbm, t_ref, o_ref, p_v, sem):
        pltpu.async_copy(p_hbm, p_v, sem).wait()
        ax = jnp.sum(p_v[...], axis=0)
        t = t_ref[...].reshape(N)
        res = t - ax
        ss_res = jnp.sum(res * res)
        ss_t = jnp.sum(t * t)
        val = jnp.sqrt(ss_res) / (jnp.sqrt(ss_t) + EPS)
        o_ref[...] = jnp.full((1, 1), val, jnp.float32)

    return pl.pallas_call(
        body,
        in_specs=[pl.BlockSpec(memory_space=pl.ANY),
                  pl.BlockSpec(memory_space=pltpu.VMEM)],
        out_specs=pl.BlockSpec(memory_space=pltpu.VMEM),
        out_shape=jax.ShapeDtypeStruct((1, 1), jnp.float32),
        scratch_shapes=[pltpu.VMEM((NUM_WORKERS, N), jnp.float32),
                        pltpu.SemaphoreType.DMA],
    )(partials, target)


def kernel(preds, target_b, matrix_vals, matrix_rows, matrix_cols, batch_map):
    nnz = matrix_vals.shape[0]
    full = nnz // BLOCK
    tail = nnz - full * BLOCK
    aux_vals = jnp.zeros((2 * BLOCK,), jnp.float32)
    aux_rows = jnp.full((2 * BLOCK,), N - 1, jnp.int32)
    aux_cols = jnp.zeros((2 * BLOCK,), jnp.int32)
    if tail:
        aux_vals = aux_vals.at[:tail].set(matrix_vals[full * BLOCK:])
        aux_rows = aux_rows.at[:tail].set(matrix_rows[full * BLOCK:])
        aux_cols = aux_cols.at[:tail].set(matrix_cols[full * BLOCK:])
    partials = _sc_partial_spmv(preds, matrix_vals, matrix_rows, matrix_cols,
                                aux_vals, aux_rows, aux_cols, full)
    out = _finish(partials, target_b.reshape(128, 128))
    return out[0, 0]
